# Initial kernel scaffold; baseline (speedup 1.0000x reference)
import jax
import jax.numpy as jnp
from jax.experimental import pallas as pl


def kernel(h, W_ref, a_ref, W_dir, a_dir, ref_neighbors, dir_neighbors):
    raise NotImplementedError("write your pallas kernel here")


# speedup vs baseline: 1.8247x; 1.8247x over previous
import functools

import jax
import jax.numpy as jnp
from jax import lax
from jax.experimental import pallas as pl
from jax.experimental.pallas import tpu as pltpu
from jax.experimental.pallas import tpu_sc as plsc

N = 10000
NPAD = 10240
IN = 128
OUT = 32
D4 = 4
RK = 10
DK = 16
NEG = 0.2

NC = 2
NS = 16
NW = NC * NS
NT = NPAD // NW
NCH = 4
CH = NT // NCH
GR = CH // 16
RB = (D4 * CH + 127) // 128
DB = (DK * CH) // 128


def _leaky(x):
    return jnp.where(x >= 0, x, NEG * x)


BN = 512


def _tc_body(h_ref, wc_ref, bm_ref, wh_ref, whd_ref, st_ref):
    P = jnp.dot(h_ref[...], wc_ref[...], preferred_element_type=jnp.float32,
                precision=lax.Precision.HIGHEST)
    wh_ref[...] = P[:, :OUT]
    whd_ref[...] = P[:, OUT:]
    st_ref[...] = lax.dot_general(bm_ref[...], P, (((0,), (1,)), ((), ())),
                                  preferred_element_type=jnp.float32,
                                  precision=lax.Precision.HIGHEST)


def _tc_project(h_pad, Wc, Bmat):
    return pl.pallas_call(
        _tc_body,
        grid=(NPAD // BN,),
        in_specs=[
            pl.BlockSpec((BN, IN), lambda i: (i, 0)),
            pl.BlockSpec((IN, 2 * OUT), lambda i: (0, 0)),
            pl.BlockSpec((2 * OUT, 8), lambda i: (0, 0)),
        ],
        out_specs=[
            pl.BlockSpec((BN, OUT), lambda i: (i, 0)),
            pl.BlockSpec((BN, OUT), lambda i: (i, 0)),
            pl.BlockSpec((8, BN), lambda i: (0, i)),
        ],
        out_shape=[
            jax.ShapeDtypeStruct((NPAD, OUT), jnp.float32),
            jax.ShapeDtypeStruct((NPAD, OUT), jnp.float32),
            jax.ShapeDtypeStruct((8, NPAD), jnp.float32),
        ],
    )(h_pad, Wc, Bmat)


def _make_sc_kernel():
    mesh = plsc.VectorSubcoreMesh(core_axis_name="c", subcore_axis_name="s",
                                  num_cores=NC, num_subcores=NS)
    scratch = [
        pltpu.VMEM((NPAD,), jnp.float32),
        pltpu.VMEM((NPAD,), jnp.float32),
        pltpu.VMEM((NT,), jnp.float32),
        pltpu.VMEM((NT,), jnp.float32),
        pltpu.VMEM((NCH * RK * RB * 128,), jnp.int32),
        pltpu.VMEM((NCH * DB * 128,), jnp.int32),
        pltpu.VMEM((RB * 128, OUT), jnp.float32),
        pltpu.VMEM((DB * 128, OUT), jnp.float32),
        pltpu.VMEM((CH, OUT), jnp.float32),
        pltpu.SemaphoreType.DMA,
        pltpu.SemaphoreType.DMA,
    ]

    @functools.partial(
        pl.kernel,
        out_type=jax.ShapeDtypeStruct((NPAD, OUT), jnp.float32),
        mesh=mesh,
        scratch_types=scratch,
        compiler_params=pltpu.CompilerParams(needs_layout_passes=False,
                                             use_tc_tiling_on_sc=False),
    )
    def sc_kernel(wh_hbm, whd_hbm, st_hbm, ridx_hbm, didx_hbm, out_hbm,
                  tref_v, tdir_v, s1_v, s2_v, ridx_v, didx_v,
                  fref_v, drows_v, out_v, sem_r, sem_d):
        wid = lax.axis_index("s") * NC + lax.axis_index("c")
        base = wid * NT

        pltpu.sync_copy(st_hbm.at[pl.ds(1 * NPAD, NPAD)], tref_v)
        pltpu.sync_copy(st_hbm.at[pl.ds(3 * NPAD, NPAD)], tdir_v)
        pltpu.sync_copy(st_hbm.at[pl.ds(base, NT)], s1_v)
        pltpu.sync_copy(st_hbm.at[pl.ds(2 * NPAD + base, NT)], s2_v)
        pltpu.sync_copy(ridx_hbm.at[pl.ds(wid * (NCH * RK * RB * 128),
                                          NCH * RK * RB * 128)], ridx_v)
        pltpu.sync_copy(didx_hbm.at[pl.ds(wid * (NCH * DB * 128),
                                          NCH * DB * 128)], didx_v)

        def chunk_body(c, carry):
            dcps = [pltpu.async_copy(whd_hbm.at[didx_v.at[pl.ds((c * DB + j) * 128, 128)]],
                                     drows_v.at[pl.ds(j * 128, 128)], sem_d)
                    for j in range(DB)]
            for k in range(RK):
                cps = [pltpu.async_copy(wh_hbm.at[ridx_v.at[pl.ds(((c * RK + k) * RB + j) * 128, 128)]],
                                        fref_v.at[pl.ds(j * 128, 128)],
                                        sem_r, add=(k > 0))
                       for j in range(RB)]
                for cp in cps:
                    cp.wait()
            for cp in dcps:
                cp.wait()

            def group_body(g, carry2):
                coff = c * CH
                s1 = s1_v[pl.ds(coff + g * 16, 16)]
                es = []
                for d in range(D4):
                    acc = plsc.load_gather(
                        tref_v,
                        [ridx_v[pl.ds((c * RK + 0) * RB * 128
                                      + d * CH + g * 16, 16)]])
                    for k in range(1, RK):
                        acc = acc + plsc.load_gather(
                            tref_v,
                            [ridx_v[pl.ds((c * RK + k) * RB * 128
                                          + d * CH + g * 16, 16)]])
                    es.append(_leaky(s1 + (1.0 / RK) * acc))
                m = jnp.maximum(jnp.maximum(es[0], es[1]),
                                jnp.maximum(es[2], es[3]))
                ex = [jnp.exp(e - m) for e in es]
                inv = (1.0 / RK) / ((ex[0] + ex[1]) + (ex[2] + ex[3]))
                alv = [e * inv for e in ex]

                s2 = s2_v[pl.ds(coff + g * 16, 16)]
                eds = []
                for k in range(DK):
                    eds.append(_leaky(s2 + plsc.load_gather(
                        tdir_v,
                        [didx_v[pl.ds(c * DB * 128 + k * CH + g * 16, 16)]])))
                m2 = functools.reduce(jnp.maximum, eds)
                ex2 = [jnp.exp(e - m2) for e in eds]
                inv2 = 1.0 / functools.reduce(lambda a, b: a + b, ex2)
                adv = [e * inv2 for e in ex2]

                nvec = g * 16 + lax.iota(jnp.int32, 16)
                for cc in range(OUT):
                    ccv = jnp.full((16,), cc, jnp.int32)
                    accA = alv[0] * plsc.load_gather(fref_v, [nvec, ccv])
                    for d in range(1, D4):
                        accA = accA + alv[d] * plsc.load_gather(
                            fref_v, [d * CH + nvec, ccv])
                    accB = adv[0] * plsc.load_gather(drows_v, [nvec, ccv])
                    for k in range(1, DK):
                        accB = accB + adv[k] * plsc.load_gather(
                            drows_v, [k * CH + nvec, ccv])
                    rA = 1.0 / (1.0 + jnp.exp(-accA))
                    rB = 1.0 / (1.0 + jnp.exp(-accB))
                    plsc.store_scatter(out_v, [nvec, ccv], 0.5 * (rA + rB))
                return carry2

            lax.fori_loop(0, GR, group_body, 0)
            pltpu.sync_copy(out_v, out_hbm.at[pl.ds(base + c * CH, CH)])
            return carry

        lax.fori_loop(0, NCH, chunk_body, 0)

    return sc_kernel


_sc_kernel = _make_sc_kernel()


def kernel(h, W_ref, a_ref, W_dir, a_dir, ref_neighbors, dir_neighbors):
    h_pad = jnp.pad(h, ((0, NPAD - N), (0, 0)))
    Wc = jnp.concatenate([W_ref, W_dir], axis=1)
    a1 = a_ref[:OUT, 0]
    a2 = a_ref[OUT:, 0]
    b1 = a_dir[:OUT, 0]
    b2 = a_dir[OUT:, 0]
    z = jnp.zeros((OUT,), jnp.float32)
    Bmat = jnp.stack(
        [jnp.concatenate([a1, z]), jnp.concatenate([a2, z]),
         jnp.concatenate([z, b1]), jnp.concatenate([z, b2]),
         jnp.zeros((2 * OUT,), jnp.float32), jnp.zeros((2 * OUT,), jnp.float32),
         jnp.zeros((2 * OUT,), jnp.float32), jnp.zeros((2 * OUT,), jnp.float32)],
        axis=1)

    wh, whd, st = _tc_project(h_pad, Wc, Bmat)

    rn = jnp.pad(ref_neighbors.astype(jnp.int32),
                 ((0, NPAD - N), (0, 0), (0, 0)))
    ridx = rn.reshape(NW, NCH, CH, D4, RK).transpose(0, 1, 4, 3, 2)\
        .reshape(NW, NCH, RK, D4 * CH)
    ridx = jnp.pad(ridx, ((0, 0), (0, 0), (0, 0), (0, RB * 128 - D4 * CH)))\
        .reshape(NW * NCH * RK * RB * 128)
    dn = jnp.pad(dir_neighbors.astype(jnp.int32), ((0, NPAD - N), (0, 0)))
    didx = dn.reshape(NW, NCH, CH, DK).transpose(0, 1, 3, 2)\
        .reshape(NW * NCH * DB * 128)

    out_pad = _sc_kernel(wh, whd, st.reshape(8 * NPAD), ridx, didx)
    return out_pad[:N]


# speedup vs baseline: 4.8692x; 2.6685x over previous
import functools

import jax
import jax.numpy as jnp
from jax import lax
from jax.experimental import pallas as pl
from jax.experimental.pallas import tpu as pltpu
from jax.experimental.pallas import tpu_sc as plsc

N = 10000
NPAD = 10240
IN = 128
OUT = 32
D4 = 4
RK = 10
DK = 16
NEG = 0.2

NC = 2
NS = 16
NW = NC * NS
NT = NPAD // NW
NG = NT // 16
DCH = 32
NDCH = NT // DCH
DP = NDCH // 2
RLEN = D4 * NT
DLEN = DK * DCH


def _leaky(x):
    return jnp.where(x >= 0, x, NEG * x)


def _sigmoid(x):
    return 1.0 / (1.0 + jnp.exp(-x))


BN = 512


def _tc_body(h_ref, wc_ref, bm_ref, wh_ref, whd_ref, st_ref):
    P = jnp.dot(h_ref[...], wc_ref[...], preferred_element_type=jnp.float32,
                precision=lax.Precision.HIGHEST)
    wh_ref[...] = P[:, :OUT]
    whd_ref[...] = P[:, OUT:]
    st_ref[...] = lax.dot_general(bm_ref[...], P, (((0,), (1,)), ((), ())),
                                  preferred_element_type=jnp.float32,
                                  precision=lax.Precision.HIGHEST)


def _tc_project(h_pad, Wc, Bmat):
    return pl.pallas_call(
        _tc_body,
        grid=(NPAD // BN,),
        in_specs=[
            pl.BlockSpec((BN, IN), lambda i: (i, 0)),
            pl.BlockSpec((IN, 2 * OUT), lambda i: (0, 0)),
            pl.BlockSpec((2 * OUT, 8), lambda i: (0, 0)),
        ],
        out_specs=[
            pl.BlockSpec((BN, OUT), lambda i: (i, 0)),
            pl.BlockSpec((BN, OUT), lambda i: (i, 0)),
            pl.BlockSpec((8, BN), lambda i: (0, i)),
        ],
        out_shape=[
            jax.ShapeDtypeStruct((NPAD, OUT), jnp.float32),
            jax.ShapeDtypeStruct((NPAD, OUT), jnp.float32),
            jax.ShapeDtypeStruct((8, NPAD), jnp.float32),
        ],
    )(h_pad, Wc, Bmat)


def _make_sc_kernel():
    mesh = plsc.VectorSubcoreMesh(core_axis_name="c", subcore_axis_name="s",
                                  num_cores=NC, num_subcores=NS)
    scratch = [
        pltpu.VMEM((NPAD,), jnp.float32),
        pltpu.VMEM((NPAD,), jnp.float32),
        pltpu.VMEM((NT,), jnp.float32),
        pltpu.VMEM((NT,), jnp.float32),
        pltpu.VMEM((RK * RLEN,), jnp.int32),
        pltpu.VMEM((NT * DK,), jnp.int32),
        pltpu.VMEM((RLEN, OUT), jnp.float32),
        pltpu.VMEM((DLEN, OUT), jnp.float32),
        pltpu.VMEM((DLEN, OUT), jnp.float32),
        pltpu.VMEM((D4, NT), jnp.float32),
        pltpu.VMEM((DK, NT), jnp.float32),
        pltpu.VMEM((DCH, OUT), jnp.float32),
        pltpu.SemaphoreType.DMA,
        pltpu.SemaphoreType.DMA,
        pltpu.SemaphoreType.DMA,
    ]

    @functools.partial(
        pl.kernel,
        out_type=jax.ShapeDtypeStruct((NPAD, OUT), jnp.float32),
        mesh=mesh,
        scratch_types=scratch,
        compiler_params=pltpu.CompilerParams(needs_layout_passes=False,
                                             use_tc_tiling_on_sc=False),
    )
    def sc_kernel(wh_hbm, whd_hbm, st_hbm, ridx_hbm, didx_hbm, out_hbm,
                  tref_v, tdir_v, s1_v, s2_v, ridx_v, didx_v,
                  fref_v, drA, drB, alpha_v, ad_v, out_v,
                  sem_r, sem_da, sem_db):
        wid = lax.axis_index("s") * NC + lax.axis_index("c")
        base = wid * NT

        pltpu.sync_copy(st_hbm.at[pl.ds(1 * NPAD, NPAD)], tref_v)
        pltpu.sync_copy(st_hbm.at[pl.ds(3 * NPAD, NPAD)], tdir_v)
        pltpu.sync_copy(st_hbm.at[pl.ds(base, NT)], s1_v)
        pltpu.sync_copy(st_hbm.at[pl.ds(2 * NPAD + base, NT)], s2_v)
        pltpu.sync_copy(ridx_hbm.at[pl.ds(wid * (RK * RLEN), RK * RLEN)],
                        ridx_v)
        pltpu.sync_copy(didx_hbm.at[pl.ds(wid * (NT * DK), NT * DK)], didx_v)

        def attn_group(g, carry):
            goff = g * 16
            s1 = s1_v[pl.ds(goff, 16)]
            es = []
            for d in range(D4):
                acc = plsc.load_gather(
                    tref_v, [ridx_v[pl.ds(d * NT + goff, 16)]])
                for k in range(1, RK):
                    acc = acc + plsc.load_gather(
                        tref_v, [ridx_v[pl.ds(k * RLEN + d * NT + goff, 16)]])
                es.append(_leaky(s1 + (1.0 / RK) * acc))
            m = jnp.maximum(jnp.maximum(es[0], es[1]),
                            jnp.maximum(es[2], es[3]))
            ex = [jnp.exp(e - m) for e in es]
            inv = (1.0 / RK) / ((ex[0] + ex[1]) + (ex[2] + ex[3]))
            for d in range(D4):
                alpha_v[d, pl.ds(goff, 16)] = ex[d] * inv

            s2 = s2_v[pl.ds(goff, 16)]
            doff = (g >> 1) * (DK * DCH) + (g & 1) * 16
            eds = []
            for k in range(DK):
                eds.append(_leaky(s2 + plsc.load_gather(
                    tdir_v, [didx_v[pl.ds(doff + k * DCH, 16)]])))
            m2 = functools.reduce(jnp.maximum, eds)
            ex2 = [jnp.exp(e - m2) for e in eds]
            inv2 = 1.0 / functools.reduce(lambda a, b: a + b, ex2)
            for k in range(DK):
                ad_v[k, pl.ds(goff, 16)] = ex2[k] * inv2
            return carry

        cp = pltpu.async_copy(wh_hbm.at[ridx_v.at[pl.ds(0, RLEN)]],
                              fref_v, sem_r)
        for k in range(RK):
            lax.fori_loop(2 * k, 2 * k + 2, attn_group, 0)
            cp.wait()
            if k + 1 < RK:
                cp = pltpu.async_copy(
                    wh_hbm.at[ridx_v.at[pl.ds((k + 1) * RLEN, RLEN)]],
                    fref_v, sem_r, add=True)

        def compute_chunk(c, drows):
            def wsum_group(gg, carry):
                noff = c * DCH + gg * 16
                nloc = gg * 16 + lax.iota(jnp.int32, 16)
                ntile = c * DCH + nloc
                alv = [alpha_v[d, pl.ds(noff, 16)] for d in range(D4)]
                adv = [ad_v[k, pl.ds(noff, 16)] for k in range(DK)]
                for cc in range(OUT):
                    ccv = jnp.full((16,), cc, jnp.int32)
                    accA = alv[0] * plsc.load_gather(fref_v, [ntile, ccv])
                    for d in range(1, D4):
                        accA = accA + alv[d] * plsc.load_gather(
                            fref_v, [d * NT + ntile, ccv])
                    accB = adv[0] * plsc.load_gather(drows, [nloc, ccv])
                    for k in range(1, DK):
                        accB = accB + adv[k] * plsc.load_gather(
                            drows, [k * DCH + nloc, ccv])
                    plsc.store_scatter(out_v, [nloc, ccv],
                                       0.5 * (_sigmoid(accA) + _sigmoid(accB)))
                return carry

            lax.fori_loop(0, DCH // 16, wsum_group, 0)
            pltpu.sync_copy(out_v, out_hbm.at[pl.ds(base + c * DCH, DCH)])

        def pair_body(p, carry):
            cA = 2 * p
            cB = 2 * p + 1
            cpA = pltpu.async_copy(
                whd_hbm.at[didx_v.at[pl.ds(cA * DLEN, DLEN)]], drA, sem_da)
            cpB = pltpu.async_copy(
                whd_hbm.at[didx_v.at[pl.ds(cB * DLEN, DLEN)]], drB, sem_db)
            cpA.wait()
            compute_chunk(cA, drA)
            cpB.wait()
            compute_chunk(cB, drB)
            return carry

        lax.fori_loop(0, DP, pair_body, 0)

    return sc_kernel


_sc_kernel = _make_sc_kernel()


def kernel(h, W_ref, a_ref, W_dir, a_dir, ref_neighbors, dir_neighbors):
    h_pad = jnp.pad(h, ((0, NPAD - N), (0, 0)))
    Wc = jnp.concatenate([W_ref, W_dir], axis=1)
    a1 = a_ref[:OUT, 0]
    a2 = a_ref[OUT:, 0]
    b1 = a_dir[:OUT, 0]
    b2 = a_dir[OUT:, 0]
    z = jnp.zeros((OUT,), jnp.float32)
    Bmat = jnp.stack(
        [jnp.concatenate([a1, z]), jnp.concatenate([a2, z]),
         jnp.concatenate([z, b1]), jnp.concatenate([z, b2]),
         jnp.zeros((2 * OUT,), jnp.float32), jnp.zeros((2 * OUT,), jnp.float32),
         jnp.zeros((2 * OUT,), jnp.float32), jnp.zeros((2 * OUT,), jnp.float32)],
        axis=1)

    wh, whd, st = _tc_project(h_pad, Wc, Bmat)

    rn = jnp.pad(ref_neighbors.astype(jnp.int32),
                 ((0, NPAD - N), (0, 0), (0, 0)))
    ridx = rn.reshape(NW, NT, D4, RK).transpose(0, 3, 2, 1).reshape(-1)
    dn = jnp.pad(dir_neighbors.astype(jnp.int32), ((0, NPAD - N), (0, 0)))
    didx = dn.reshape(NW, NDCH, DCH, DK).transpose(0, 1, 3, 2).reshape(-1)

    out_pad = _sc_kernel(wh, whd, st.reshape(8 * NPAD), ridx, didx)
    return out_pad[:N]


# speedup vs baseline: 12.2713x; 2.5202x over previous
import functools

import jax
import jax.numpy as jnp
from jax import lax
from jax.experimental import pallas as pl
from jax.experimental.pallas import tpu as pltpu
from jax.experimental.pallas import tpu_sc as plsc

N = 10000
NPAD = 10240
IN = 128
OUT = 32
D4 = 4
RK = 10
DK = 16
NEG = 0.2

NC = 2
NS = 16
NW = NC * NS
NT = NPAD // NW
NG = NT // 16
DCH = 32
NDCH = NT // DCH
RLEN = D4 * NT
CP = OUT // 2


def _leaky(x):
    return jnp.where(x >= 0, x, NEG * x)


def _sigmoid(x):
    return 1.0 / (1.0 + jnp.exp(-x))


BN = 512


def _tc_body(h_ref, wc_ref, bm_ref, whT_ref, whdT_ref, st_ref):
    PT = lax.dot_general(wc_ref[...], h_ref[...], (((0,), (1,)), ((), ())),
                         preferred_element_type=jnp.float32,
                         precision=lax.Precision.HIGHEST)
    whT_ref[...] = PT[:OUT]
    whdT_ref[...] = PT[OUT:]
    st_ref[...] = lax.dot_general(bm_ref[...], PT, (((0,), (0,)), ((), ())),
                                  preferred_element_type=jnp.float32,
                                  precision=lax.Precision.HIGHEST)


def _tc_project(h_pad, Wc, Bmat):
    return pl.pallas_call(
        _tc_body,
        grid=(NPAD // BN,),
        in_specs=[
            pl.BlockSpec((BN, IN), lambda i: (i, 0)),
            pl.BlockSpec((IN, 2 * OUT), lambda i: (0, 0)),
            pl.BlockSpec((2 * OUT, 8), lambda i: (0, 0)),
        ],
        out_specs=[
            pl.BlockSpec((OUT, BN), lambda i: (0, i)),
            pl.BlockSpec((OUT, BN), lambda i: (0, i)),
            pl.BlockSpec((8, BN), lambda i: (0, i)),
        ],
        out_shape=[
            jax.ShapeDtypeStruct((OUT, NPAD), jnp.float32),
            jax.ShapeDtypeStruct((OUT, NPAD), jnp.float32),
            jax.ShapeDtypeStruct((8, NPAD), jnp.float32),
        ],
    )(h_pad, Wc, Bmat)


def _make_sc_kernel():
    mesh = plsc.VectorSubcoreMesh(core_axis_name="c", subcore_axis_name="s",
                                  num_cores=NC, num_subcores=NS)
    scratch = [
        pltpu.VMEM((NPAD,), jnp.float32),
        pltpu.VMEM((NPAD,), jnp.float32),
        pltpu.VMEM((NT,), jnp.float32),
        pltpu.VMEM((NT,), jnp.float32),
        pltpu.VMEM((RK * RLEN,), jnp.int32),
        pltpu.VMEM((NT * DK,), jnp.int32),
        pltpu.VMEM((NPAD,), jnp.float32),
        pltpu.VMEM((NPAD,), jnp.float32),
        pltpu.VMEM((NPAD,), jnp.float32),
        pltpu.VMEM((NPAD,), jnp.float32),
        pltpu.VMEM((D4, NT), jnp.float32),
        pltpu.VMEM((DK, NT), jnp.float32),
        pltpu.VMEM((NT * OUT,), jnp.float32),
        pltpu.SemaphoreType.DMA,
        pltpu.SemaphoreType.DMA,
    ]

    @functools.partial(
        pl.kernel,
        out_type=jax.ShapeDtypeStruct((NPAD * OUT,), jnp.float32),
        mesh=mesh,
        scratch_types=scratch,
        compiler_params=pltpu.CompilerParams(needs_layout_passes=False,
                                             use_tc_tiling_on_sc=False),
    )
    def sc_kernel(whT_hbm, whdT_hbm, st_hbm, ridx_hbm, didx_hbm, out_hbm,
                  tref_v, tdir_v, s1_v, s2_v, ridx_v, didx_v,
                  cwA, cdA, cwB, cdB, alpha_v, ad_v, out_v,
                  sem_a, sem_b):
        sid = lax.axis_index("s")
        wid = sid * NC + lax.axis_index("c")
        base = wid * NT

        pltpu.sync_copy(st_hbm.at[pl.ds(1 * NPAD, NPAD)], tref_v)
        pltpu.sync_copy(st_hbm.at[pl.ds(3 * NPAD, NPAD)], tdir_v)
        pltpu.sync_copy(st_hbm.at[pl.ds(base, NT)], s1_v)
        pltpu.sync_copy(st_hbm.at[pl.ds(2 * NPAD + base, NT)], s2_v)
        pltpu.sync_copy(ridx_hbm.at[pl.ds(wid * (RK * RLEN), RK * RLEN)],
                        ridx_v)
        pltpu.sync_copy(didx_hbm.at[pl.ds(wid * (NT * DK), NT * DK)], didx_v)

        pre = [pltpu.async_copy(whT_hbm.at[pl.ds(0, NPAD)], cwA, sem_a),
               pltpu.async_copy(whdT_hbm.at[pl.ds(0, NPAD)], cdA, sem_a),
               pltpu.async_copy(whT_hbm.at[pl.ds(NPAD, NPAD)], cwB, sem_b),
               pltpu.async_copy(whdT_hbm.at[pl.ds(NPAD, NPAD)], cdB, sem_b)]

        def attn_group(g, carry):
            goff = g * 16
            s1 = s1_v[pl.ds(goff, 16)]
            es = []
            for d in range(D4):
                acc = plsc.load_gather(
                    tref_v, [ridx_v[pl.ds(d * NT + goff, 16)]])
                for k in range(1, RK):
                    acc = acc + plsc.load_gather(
                        tref_v, [ridx_v[pl.ds(k * RLEN + d * NT + goff, 16)]])
                es.append(_leaky(s1 + (1.0 / RK) * acc))
            m = jnp.maximum(jnp.maximum(es[0], es[1]),
                            jnp.maximum(es[2], es[3]))
            ex = [jnp.exp(e - m) for e in es]
            inv = (1.0 / RK) / ((ex[0] + ex[1]) + (ex[2] + ex[3]))
            for d in range(D4):
                alpha_v[d, pl.ds(goff, 16)] = ex[d] * inv

            s2 = s2_v[pl.ds(goff, 16)]
            doff = (g >> 1) * (DK * DCH) + (g & 1) * 16
            eds = []
            for k in range(DK):
                eds.append(_leaky(s2 + plsc.load_gather(
                    tdir_v, [didx_v[pl.ds(doff + k * DCH, 16)]])))
            m2 = functools.reduce(jnp.maximum, eds)
            ex2 = [jnp.exp(e - m2) for e in eds]
            inv2 = 1.0 / functools.reduce(lambda a, b: a + b, ex2)
            for k in range(DK):
                ad_v[k, pl.ds(goff, 16)] = ex2[k] * inv2
            return carry

        lax.fori_loop(0, NG, attn_group, 0)

        def make_cc_compute(col_wh, col_whd):
            def gbody(g, cc):
                goff = g * 16
                nloc = goff + lax.iota(jnp.int32, 16)
                accA = None
                for d in range(D4):
                    t = plsc.load_gather(
                        col_wh, [ridx_v[pl.ds(d * NT + goff, 16)]])
                    for k in range(1, RK):
                        t = t + plsc.load_gather(
                            col_wh,
                            [ridx_v[pl.ds(k * RLEN + d * NT + goff, 16)]])
                    w = alpha_v[d, pl.ds(goff, 16)] * t
                    accA = w if accA is None else accA + w
                doff = (g >> 1) * (DK * DCH) + (g & 1) * 16
                accB = None
                for k in range(DK):
                    w = ad_v[k, pl.ds(goff, 16)] * plsc.load_gather(
                        col_whd, [didx_v[pl.ds(doff + k * DCH, 16)]])
                    accB = w if accB is None else accB + w
                val = 0.5 * (_sigmoid(accA) + _sigmoid(accB))
                plsc.store_scatter(out_v, [nloc * OUT + cc], val)
                return cc

            return gbody

        gbody_A = make_cc_compute(cwA, cdA)
        gbody_B = make_cc_compute(cwB, cdB)

        def pair_body(p, carry):
            ccA = 2 * p
            ccB = 2 * p + 1
            pre[0].wait()
            pre[1].wait()
            lax.fori_loop(0, NG, gbody_A, ccA)

            @pl.when(p + 1 < CP)
            def _next_a():
                pltpu.async_copy(
                    whT_hbm.at[pl.ds((ccA + 2) * NPAD, NPAD)], cwA, sem_a)
                pltpu.async_copy(
                    whdT_hbm.at[pl.ds((ccA + 2) * NPAD, NPAD)], cdA, sem_a)

            pre[2].wait()
            pre[3].wait()
            lax.fori_loop(0, NG, gbody_B, ccB)

            @pl.when(p + 1 < CP)
            def _next_b():
                pltpu.async_copy(
                    whT_hbm.at[pl.ds((ccB + 2) * NPAD, NPAD)], cwB, sem_b)
                pltpu.async_copy(
                    whdT_hbm.at[pl.ds((ccB + 2) * NPAD, NPAD)], cdB, sem_b)

            return carry

        lax.fori_loop(0, CP, pair_body, 0)

        pltpu.sync_copy(out_v, out_hbm.at[pl.ds(base * OUT, NT * OUT)])

    return sc_kernel


_sc_kernel = _make_sc_kernel()


def kernel(h, W_ref, a_ref, W_dir, a_dir, ref_neighbors, dir_neighbors):
    h_pad = jnp.pad(h, ((0, NPAD - N), (0, 0)))
    Wc = jnp.concatenate([W_ref, W_dir], axis=1)
    a1 = a_ref[:OUT, 0]
    a2 = a_ref[OUT:, 0]
    b1 = a_dir[:OUT, 0]
    b2 = a_dir[OUT:, 0]
    z = jnp.zeros((OUT,), jnp.float32)
    Bmat = jnp.stack(
        [jnp.concatenate([a1, z]), jnp.concatenate([a2, z]),
         jnp.concatenate([z, b1]), jnp.concatenate([z, b2]),
         jnp.zeros((2 * OUT,), jnp.float32), jnp.zeros((2 * OUT,), jnp.float32),
         jnp.zeros((2 * OUT,), jnp.float32), jnp.zeros((2 * OUT,), jnp.float32)],
        axis=1)

    whT, whdT, st = _tc_project(h_pad, Wc, Bmat)

    rn = jnp.pad(ref_neighbors.astype(jnp.int32),
                 ((0, NPAD - N), (0, 0), (0, 0)))
    ridx = rn.reshape(NW, NT, D4, RK).transpose(0, 3, 2, 1).reshape(-1)
    dn = jnp.pad(dir_neighbors.astype(jnp.int32), ((0, NPAD - N), (0, 0)))
    didx = dn.reshape(NW, NDCH, DCH, DK).transpose(0, 1, 3, 2).reshape(-1)

    out_pad = _sc_kernel(whT.reshape(OUT * NPAD), whdT.reshape(OUT * NPAD),
                         st.reshape(8 * NPAD), ridx, didx)
    return out_pad.reshape(NPAD, OUT)[:N]


# speedup vs baseline: 12.5256x; 1.0207x over previous
import functools

import jax
import jax.numpy as jnp
from jax import lax
from jax.experimental import pallas as pl
from jax.experimental.pallas import tpu as pltpu
from jax.experimental.pallas import tpu_sc as plsc

N = 10000
NPAD = 10240
IN = 128
OUT = 32
D4 = 4
RK = 10
DK = 16
NEG = 0.2

NC = 2
NS = 16
NW = NC * NS
NT = NPAD // NW
NG = NT // 16
DCH = 32
NDCH = NT // DCH
RLEN = D4 * NT
CP = OUT // 2


def _leaky(x):
    return jnp.where(x >= 0, x, NEG * x)


def _sigmoid(x):
    return 1.0 / (1.0 + jnp.exp(-x))


BN = 512


def _tc_body(h_ref, wc_ref, bm_ref, whT_ref, whdT_ref, st_ref):
    PT = lax.dot_general(wc_ref[...], h_ref[...], (((0,), (1,)), ((), ())),
                         preferred_element_type=jnp.float32)
    whT_ref[...] = PT[:OUT]
    whdT_ref[...] = PT[OUT:]
    st_ref[...] = lax.dot_general(bm_ref[...], PT, (((0,), (0,)), ((), ())),
                                  preferred_element_type=jnp.float32)


def _tc_project(h_pad, Wc, Bmat):
    return pl.pallas_call(
        _tc_body,
        grid=(NPAD // BN,),
        in_specs=[
            pl.BlockSpec((BN, IN), lambda i: (i, 0)),
            pl.BlockSpec((IN, 2 * OUT), lambda i: (0, 0)),
            pl.BlockSpec((2 * OUT, 8), lambda i: (0, 0)),
        ],
        out_specs=[
            pl.BlockSpec((OUT, BN), lambda i: (0, i)),
            pl.BlockSpec((OUT, BN), lambda i: (0, i)),
            pl.BlockSpec((8, BN), lambda i: (0, i)),
        ],
        out_shape=[
            jax.ShapeDtypeStruct((OUT, NPAD), jnp.float32),
            jax.ShapeDtypeStruct((OUT, NPAD), jnp.float32),
            jax.ShapeDtypeStruct((8, NPAD), jnp.float32),
        ],
    )(h_pad, Wc, Bmat)


def _make_sc_kernel():
    mesh = plsc.VectorSubcoreMesh(core_axis_name="c", subcore_axis_name="s",
                                  num_cores=NC, num_subcores=NS)
    scratch = [
        pltpu.VMEM((NPAD,), jnp.float32),
        pltpu.VMEM((NPAD,), jnp.float32),
        pltpu.VMEM((NT,), jnp.float32),
        pltpu.VMEM((NT,), jnp.float32),
        pltpu.VMEM((RK * RLEN,), jnp.int32),
        pltpu.VMEM((NT * DK,), jnp.int32),
        pltpu.VMEM((NPAD,), jnp.float32),
        pltpu.VMEM((NPAD,), jnp.float32),
        pltpu.VMEM((NPAD,), jnp.float32),
        pltpu.VMEM((NPAD,), jnp.float32),
        pltpu.VMEM((D4, NT), jnp.float32),
        pltpu.VMEM((DK, NT), jnp.float32),
        pltpu.VMEM((NT * OUT,), jnp.float32),
        pltpu.SemaphoreType.DMA,
        pltpu.SemaphoreType.DMA,
    ]

    @functools.partial(
        pl.kernel,
        out_type=jax.ShapeDtypeStruct((NPAD * OUT,), jnp.float32),
        mesh=mesh,
        scratch_types=scratch,
        compiler_params=pltpu.CompilerParams(needs_layout_passes=False,
                                             use_tc_tiling_on_sc=False),
    )
    def sc_kernel(whT_hbm, whdT_hbm, st_hbm, ridx_hbm, didx_hbm, out_hbm,
                  tref_v, tdir_v, s1_v, s2_v, ridx_v, didx_v,
                  cwA, cdA, cwB, cdB, alpha_v, ad_v, out_v,
                  sem_a, sem_b):
        sid = lax.axis_index("s")
        wid = sid * NC + lax.axis_index("c")
        base = wid * NT

        pltpu.sync_copy(st_hbm.at[pl.ds(1 * NPAD, NPAD)], tref_v)
        pltpu.sync_copy(st_hbm.at[pl.ds(3 * NPAD, NPAD)], tdir_v)
        pltpu.sync_copy(st_hbm.at[pl.ds(base, NT)], s1_v)
        pltpu.sync_copy(st_hbm.at[pl.ds(2 * NPAD + base, NT)], s2_v)
        pltpu.sync_copy(ridx_hbm.at[pl.ds(wid * (RK * RLEN), RK * RLEN)],
                        ridx_v)
        pltpu.sync_copy(didx_hbm.at[pl.ds(wid * (NT * DK), NT * DK)], didx_v)

        pre = [pltpu.async_copy(whT_hbm.at[pl.ds(0, NPAD)], cwA, sem_a),
               pltpu.async_copy(whdT_hbm.at[pl.ds(0, NPAD)], cdA, sem_a),
               pltpu.async_copy(whT_hbm.at[pl.ds(NPAD, NPAD)], cwB, sem_b),
               pltpu.async_copy(whdT_hbm.at[pl.ds(NPAD, NPAD)], cdB, sem_b)]

        def attn_group(g, carry):
            goff = g * 16
            s1 = s1_v[pl.ds(goff, 16)]
            es = []
            for d in range(D4):
                acc = plsc.load_gather(
                    tref_v, [ridx_v[pl.ds(d * NT + goff, 16)]])
                for k in range(1, RK):
                    acc = acc + plsc.load_gather(
                        tref_v, [ridx_v[pl.ds(k * RLEN + d * NT + goff, 16)]])
                es.append(_leaky(s1 + (1.0 / RK) * acc))
            m = jnp.maximum(jnp.maximum(es[0], es[1]),
                            jnp.maximum(es[2], es[3]))
            ex = [jnp.exp(e - m) for e in es]
            inv = (1.0 / RK) / ((ex[0] + ex[1]) + (ex[2] + ex[3]))
            for d in range(D4):
                alpha_v[d, pl.ds(goff, 16)] = ex[d] * inv

            s2 = s2_v[pl.ds(goff, 16)]
            doff = (g >> 1) * (DK * DCH) + (g & 1) * 16
            eds = []
            for k in range(DK):
                eds.append(_leaky(s2 + plsc.load_gather(
                    tdir_v, [didx_v[pl.ds(doff + k * DCH, 16)]])))
            m2 = functools.reduce(jnp.maximum, eds)
            ex2 = [jnp.exp(e - m2) for e in eds]
            inv2 = 1.0 / functools.reduce(lambda a, b: a + b, ex2)
            for k in range(DK):
                ad_v[k, pl.ds(goff, 16)] = ex2[k] * inv2
            return carry

        lax.fori_loop(0, NG, attn_group, 0)

        def make_cc_compute(col_wh, col_whd):
            def gbody(g, cc):
                goff = g * 16
                nloc = goff + lax.iota(jnp.int32, 16)
                accA = None
                for d in range(D4):
                    t = plsc.load_gather(
                        col_wh, [ridx_v[pl.ds(d * NT + goff, 16)]])
                    for k in range(1, RK):
                        t = t + plsc.load_gather(
                            col_wh,
                            [ridx_v[pl.ds(k * RLEN + d * NT + goff, 16)]])
                    w = alpha_v[d, pl.ds(goff, 16)] * t
                    accA = w if accA is None else accA + w
                doff = (g >> 1) * (DK * DCH) + (g & 1) * 16
                accB = None
                for k in range(DK):
                    w = ad_v[k, pl.ds(goff, 16)] * plsc.load_gather(
                        col_whd, [didx_v[pl.ds(doff + k * DCH, 16)]])
                    accB = w if accB is None else accB + w
                val = 0.5 * (_sigmoid(accA) + _sigmoid(accB))
                plsc.store_scatter(out_v, [nloc * OUT + cc], val)
                return cc

            return gbody

        gbody_A = make_cc_compute(cwA, cdA)
        gbody_B = make_cc_compute(cwB, cdB)

        def pair_body(p, carry):
            ccA = 2 * p
            ccB = 2 * p + 1
            pre[0].wait()
            pre[1].wait()
            lax.fori_loop(0, NG, gbody_A, ccA)

            @pl.when(p + 1 < CP)
            def _next_a():
                pltpu.async_copy(
                    whT_hbm.at[pl.ds((ccA + 2) * NPAD, NPAD)], cwA, sem_a)
                pltpu.async_copy(
                    whdT_hbm.at[pl.ds((ccA + 2) * NPAD, NPAD)], cdA, sem_a)

            pre[2].wait()
            pre[3].wait()
            lax.fori_loop(0, NG, gbody_B, ccB)

            @pl.when(p + 1 < CP)
            def _next_b():
                pltpu.async_copy(
                    whT_hbm.at[pl.ds((ccB + 2) * NPAD, NPAD)], cwB, sem_b)
                pltpu.async_copy(
                    whdT_hbm.at[pl.ds((ccB + 2) * NPAD, NPAD)], cdB, sem_b)

            return carry

        lax.fori_loop(0, CP, pair_body, 0)

        pltpu.sync_copy(out_v, out_hbm.at[pl.ds(base * OUT, NT * OUT)])

    return sc_kernel


_sc_kernel = _make_sc_kernel()


def kernel(h, W_ref, a_ref, W_dir, a_dir, ref_neighbors, dir_neighbors):
    h_pad = jnp.pad(h, ((0, NPAD - N), (0, 0)))
    Wc = jnp.concatenate([W_ref, W_dir], axis=1)
    a1 = a_ref[:OUT, 0]
    a2 = a_ref[OUT:, 0]
    b1 = a_dir[:OUT, 0]
    b2 = a_dir[OUT:, 0]
    z = jnp.zeros((OUT,), jnp.float32)
    Bmat = jnp.stack(
        [jnp.concatenate([a1, z]), jnp.concatenate([a2, z]),
         jnp.concatenate([z, b1]), jnp.concatenate([z, b2]),
         jnp.zeros((2 * OUT,), jnp.float32), jnp.zeros((2 * OUT,), jnp.float32),
         jnp.zeros((2 * OUT,), jnp.float32), jnp.zeros((2 * OUT,), jnp.float32)],
        axis=1)

    whT, whdT, st = _tc_project(h_pad, Wc, Bmat)

    rn = jnp.pad(ref_neighbors.astype(jnp.int32),
                 ((0, NPAD - N), (0, 0), (0, 0)))
    ridx = rn.reshape(NW, NT, D4, RK).transpose(0, 3, 2, 1).reshape(-1)
    dn = jnp.pad(dir_neighbors.astype(jnp.int32), ((0, NPAD - N), (0, 0)))
    didx = dn.reshape(NW, NDCH, DCH, DK).transpose(0, 1, 3, 2).reshape(-1)

    out_pad = _sc_kernel(whT.reshape(OUT * NPAD), whdT.reshape(OUT * NPAD),
                         st.reshape(8 * NPAD), ridx, didx)
    return out_pad.reshape(NPAD, OUT)[:N]


# speedup vs baseline: 14.7745x; 1.1795x over previous
import functools

import jax
import jax.numpy as jnp
from jax import lax
from jax.experimental import pallas as pl
from jax.experimental.pallas import tpu as pltpu
from jax.experimental.pallas import tpu_sc as plsc

N = 10000
NPAD = 10240
IN = 128
OUT = 32
D4 = 4
RK = 10
DK = 16
NEG = 0.2

NC = 2
NS = 16
NW = NC * NS
NT = NPAD // NW
NG = NT // 16
DCH = 32
NDCH = NT // DCH
RLEN = D4 * NT
CP = OUT // 2
CPP = CP // 2


def _leaky(x):
    return jnp.where(x >= 0, x, NEG * x)


def _sigmoid(x):
    return 1.0 / (1.0 + jnp.exp(-x))


BN = 512


def _tc_body(h_ref, wc_ref, bm_ref, whT_ref, whdT_ref, st_ref):
    PT = lax.dot_general(wc_ref[...], h_ref[...], (((0,), (1,)), ((), ())),
                         preferred_element_type=jnp.float32)
    whT_ref[...] = PT[:OUT]
    whdT_ref[...] = PT[OUT:]
    st_ref[...] = lax.dot_general(bm_ref[...], PT, (((0,), (0,)), ((), ())),
                                  preferred_element_type=jnp.float32)


def _tc_project(h_pad, Wc, Bmat):
    return pl.pallas_call(
        _tc_body,
        grid=(NPAD // BN,),
        in_specs=[
            pl.BlockSpec((BN, IN), lambda i: (i, 0)),
            pl.BlockSpec((IN, 2 * OUT), lambda i: (0, 0)),
            pl.BlockSpec((2 * OUT, 8), lambda i: (0, 0)),
        ],
        out_specs=[
            pl.BlockSpec((OUT, BN), lambda i: (0, i)),
            pl.BlockSpec((OUT, BN), lambda i: (0, i)),
            pl.BlockSpec((8, BN), lambda i: (0, i)),
        ],
        out_shape=[
            jax.ShapeDtypeStruct((OUT, NPAD), jnp.float32),
            jax.ShapeDtypeStruct((OUT, NPAD), jnp.float32),
            jax.ShapeDtypeStruct((8, NPAD), jnp.float32),
        ],
    )(h_pad, Wc, Bmat)


def _make_sc_kernel():
    mesh = plsc.VectorSubcoreMesh(core_axis_name="c", subcore_axis_name="s",
                                  num_cores=NC, num_subcores=NS)
    scratch = [
        pltpu.VMEM((NPAD,), jnp.float32),
        pltpu.VMEM((NPAD,), jnp.float32),
        pltpu.VMEM((NT,), jnp.float32),
        pltpu.VMEM((NT,), jnp.float32),
        pltpu.VMEM((RK * RLEN,), jnp.int32),
        pltpu.VMEM((NT * DK,), jnp.int32),
        pltpu.VMEM((NPAD,), jnp.int32),
        pltpu.VMEM((NPAD,), jnp.int32),
        pltpu.VMEM((NPAD,), jnp.int32),
        pltpu.VMEM((NPAD,), jnp.int32),
        pltpu.VMEM((D4, NT), jnp.float32),
        pltpu.VMEM((DK, NT), jnp.float32),
        pltpu.VMEM((NT * OUT,), jnp.float32),
        pltpu.SemaphoreType.DMA,
        pltpu.SemaphoreType.DMA,
    ]

    @functools.partial(
        pl.kernel,
        out_type=jax.ShapeDtypeStruct((NPAD * OUT,), jnp.float32),
        mesh=mesh,
        scratch_types=scratch,
        compiler_params=pltpu.CompilerParams(needs_layout_passes=False,
                                             use_tc_tiling_on_sc=False),
    )
    def sc_kernel(whT_hbm, whdT_hbm, st_hbm, ridx_hbm, didx_hbm, out_hbm,
                  tref_v, tdir_v, s1_v, s2_v, ridx_v, didx_v,
                  cwA, cdA, cwB, cdB, alpha_v, ad_v, out_v,
                  sem_a, sem_b):
        sid = lax.axis_index("s")
        wid = sid * NC + lax.axis_index("c")
        base = wid * NT

        pltpu.sync_copy(st_hbm.at[pl.ds(1 * NPAD, NPAD)], tref_v)
        pltpu.sync_copy(st_hbm.at[pl.ds(3 * NPAD, NPAD)], tdir_v)
        pltpu.sync_copy(st_hbm.at[pl.ds(base, NT)], s1_v)
        pltpu.sync_copy(st_hbm.at[pl.ds(2 * NPAD + base, NT)], s2_v)
        pltpu.sync_copy(ridx_hbm.at[pl.ds(wid * (RK * RLEN), RK * RLEN)],
                        ridx_v)
        pltpu.sync_copy(didx_hbm.at[pl.ds(wid * (NT * DK), NT * DK)], didx_v)

        pre = [pltpu.async_copy(whT_hbm.at[pl.ds(0, NPAD)], cwA, sem_a),
               pltpu.async_copy(whdT_hbm.at[pl.ds(0, NPAD)], cdA, sem_a),
               pltpu.async_copy(whT_hbm.at[pl.ds(NPAD, NPAD)], cwB, sem_b),
               pltpu.async_copy(whdT_hbm.at[pl.ds(NPAD, NPAD)], cdB, sem_b)]

        def attn_group(g, carry):
            goff = g * 16
            s1 = s1_v[pl.ds(goff, 16)]
            es = []
            for d in range(D4):
                acc = plsc.load_gather(
                    tref_v, [ridx_v[pl.ds(d * NT + goff, 16)]])
                for k in range(1, RK):
                    acc = acc + plsc.load_gather(
                        tref_v, [ridx_v[pl.ds(k * RLEN + d * NT + goff, 16)]])
                es.append(_leaky(s1 + (1.0 / RK) * acc))
            m = jnp.maximum(jnp.maximum(es[0], es[1]),
                            jnp.maximum(es[2], es[3]))
            ex = [jnp.exp(e - m) for e in es]
            inv = (1.0 / RK) / ((ex[0] + ex[1]) + (ex[2] + ex[3]))
            for d in range(D4):
                alpha_v[d, pl.ds(goff, 16)] = ex[d] * inv

            s2 = s2_v[pl.ds(goff, 16)]
            doff = (g >> 1) * (DK * DCH) + (g & 1) * 16
            eds = []
            for k in range(DK):
                eds.append(_leaky(s2 + plsc.load_gather(
                    tdir_v, [didx_v[pl.ds(doff + k * DCH, 16)]])))
            m2 = functools.reduce(jnp.maximum, eds)
            ex2 = [jnp.exp(e - m2) for e in eds]
            inv2 = 1.0 / functools.reduce(lambda a, b: a + b, ex2)
            for k in range(DK):
                ad_v[k, pl.ds(goff, 16)] = ex2[k] * inv2
            return carry

        lax.fori_loop(0, NG, attn_group, 0)

        def make_cc_compute(col_wh, col_whd):
            def gbody(g, pr):
                goff = g * 16
                nloc = goff + lax.iota(jnp.int32, 16)
                cc_lo = 2 * pr
                accA_lo = accA_hi = None
                for d in range(D4):
                    ta = tb = None
                    for k in range(RK):
                        g32 = plsc.load_gather(
                            col_wh,
                            [ridx_v[pl.ds(k * RLEN + d * NT + goff, 16)]])
                        ua, ub = plsc.unpack(
                            plsc.bitcast(g32, jnp.bfloat16),
                            format=plsc.PackFormat.INTERLEAVED)
                        ta = ua if ta is None else ta + ua
                        tb = ub if tb is None else tb + ub
                    w = alpha_v[d, pl.ds(goff, 16)]
                    wa = w * ta
                    wb = w * tb
                    accA_lo = wa if accA_lo is None else accA_lo + wa
                    accA_hi = wb if accA_hi is None else accA_hi + wb
                doff = (g >> 1) * (DK * DCH) + (g & 1) * 16
                accB_lo = accB_hi = None
                for k in range(DK):
                    g32 = plsc.load_gather(
                        col_whd, [didx_v[pl.ds(doff + k * DCH, 16)]])
                    ua, ub = plsc.unpack(
                        plsc.bitcast(g32, jnp.bfloat16),
                        format=plsc.PackFormat.INTERLEAVED)
                    w = ad_v[k, pl.ds(goff, 16)]
                    wa = w * ua
                    wb = w * ub
                    accB_lo = wa if accB_lo is None else accB_lo + wa
                    accB_hi = wb if accB_hi is None else accB_hi + wb
                val_lo = 0.5 * (_sigmoid(accA_lo) + _sigmoid(accB_lo))
                val_hi = 0.5 * (_sigmoid(accA_hi) + _sigmoid(accB_hi))
                plsc.store_scatter(out_v, [nloc * OUT + cc_lo], val_lo)
                plsc.store_scatter(out_v, [nloc * OUT + cc_lo + 1], val_hi)
                return pr

            return gbody

        gbody_A = make_cc_compute(cwA, cdA)
        gbody_B = make_cc_compute(cwB, cdB)

        def pair_body(p, carry):
            prA = 2 * p
            prB = 2 * p + 1
            pre[0].wait()
            pre[1].wait()
            lax.fori_loop(0, NG, gbody_A, prA)

            @pl.when(p + 1 < CPP)
            def _next_a():
                pltpu.async_copy(
                    whT_hbm.at[pl.ds((prA + 2) * NPAD, NPAD)], cwA, sem_a)
                pltpu.async_copy(
                    whdT_hbm.at[pl.ds((prA + 2) * NPAD, NPAD)], cdA, sem_a)

            pre[2].wait()
            pre[3].wait()
            lax.fori_loop(0, NG, gbody_B, prB)

            @pl.when(p + 1 < CPP)
            def _next_b():
                pltpu.async_copy(
                    whT_hbm.at[pl.ds((prB + 2) * NPAD, NPAD)], cwB, sem_b)
                pltpu.async_copy(
                    whdT_hbm.at[pl.ds((prB + 2) * NPAD, NPAD)], cdB, sem_b)

            return carry

        lax.fori_loop(0, CPP, pair_body, 0)

        pltpu.sync_copy(out_v, out_hbm.at[pl.ds(base * OUT, NT * OUT)])

    return sc_kernel


_sc_kernel = _make_sc_kernel()


def kernel(h, W_ref, a_ref, W_dir, a_dir, ref_neighbors, dir_neighbors):
    h_pad = jnp.pad(h, ((0, NPAD - N), (0, 0)))
    Wc = jnp.concatenate([W_ref, W_dir], axis=1)
    a1 = a_ref[:OUT, 0]
    a2 = a_ref[OUT:, 0]
    b1 = a_dir[:OUT, 0]
    b2 = a_dir[OUT:, 0]
    z = jnp.zeros((OUT,), jnp.float32)
    Bmat = jnp.stack(
        [jnp.concatenate([a1, z]), jnp.concatenate([a2, z]),
         jnp.concatenate([z, b1]), jnp.concatenate([z, b2]),
         jnp.zeros((2 * OUT,), jnp.float32), jnp.zeros((2 * OUT,), jnp.float32),
         jnp.zeros((2 * OUT,), jnp.float32), jnp.zeros((2 * OUT,), jnp.float32)],
        axis=1)

    whT, whdT, st = _tc_project(h_pad, Wc, Bmat)

    rn = jnp.pad(ref_neighbors.astype(jnp.int32),
                 ((0, NPAD - N), (0, 0), (0, 0)))
    ridx = rn.reshape(NW, NT, D4, RK).transpose(0, 3, 2, 1).reshape(-1)
    dn = jnp.pad(dir_neighbors.astype(jnp.int32), ((0, NPAD - N), (0, 0)))
    didx = dn.reshape(NW, NDCH, DCH, DK).transpose(0, 1, 3, 2).reshape(-1)

    def _pack_pairs(t):
        tb = t.astype(jnp.bfloat16)
        return lax.bitcast_convert_type(
            jnp.stack([tb[0::2], tb[1::2]], axis=-1), jnp.int32).reshape(-1)

    out_pad = _sc_kernel(_pack_pairs(whT), _pack_pairs(whdT),
                         st.reshape(8 * NPAD), ridx, didx)
    return out_pad.reshape(NPAD, OUT)[:N]


# speedup vs baseline: 17.4226x; 1.1792x over previous
import functools

import jax
import jax.numpy as jnp
from jax import lax
from jax.experimental import pallas as pl
from jax.experimental.pallas import tpu as pltpu
from jax.experimental.pallas import tpu_sc as plsc

N = 10000
NPAD = 10240
IN = 128
OUT = 32
D4 = 4
RK = 10
DK = 16
NEG = 0.2

NC = 2
NS = 16
NW = NC * NS
NT = NPAD // NW
NG = NT // 16
DCH = 32
NDCH = NT // DCH
RLEN = D4 * NT
NQ = OUT // 4
QI = NQ // 2


def _leaky(x):
    return jnp.where(x >= 0, x, NEG * x)


def _sigmoid(x):
    return 1.0 / (1.0 + jnp.exp(-x))


BN = 512


def _tc_body(h_ref, wc_ref, bm_ref, whT_ref, whdT_ref, st_ref):
    PT = lax.dot_general(wc_ref[...], h_ref[...], (((0,), (1,)), ((), ())),
                         preferred_element_type=jnp.float32)
    whT_ref[...] = PT[:OUT]
    whdT_ref[...] = PT[OUT:]
    st_ref[...] = lax.dot_general(bm_ref[...], PT, (((0,), (0,)), ((), ())),
                                  preferred_element_type=jnp.float32)


def _tc_project(h_pad, Wc, Bmat):
    return pl.pallas_call(
        _tc_body,
        grid=(NPAD // BN,),
        in_specs=[
            pl.BlockSpec((BN, IN), lambda i: (i, 0)),
            pl.BlockSpec((IN, 2 * OUT), lambda i: (0, 0)),
            pl.BlockSpec((2 * OUT, 8), lambda i: (0, 0)),
        ],
        out_specs=[
            pl.BlockSpec((OUT, BN), lambda i: (0, i)),
            pl.BlockSpec((OUT, BN), lambda i: (0, i)),
            pl.BlockSpec((8, BN), lambda i: (0, i)),
        ],
        out_shape=[
            jax.ShapeDtypeStruct((OUT, NPAD), jnp.float32),
            jax.ShapeDtypeStruct((OUT, NPAD), jnp.float32),
            jax.ShapeDtypeStruct((8, NPAD), jnp.float32),
        ],
    )(h_pad, Wc, Bmat)


def _make_sc_kernel():
    mesh = plsc.VectorSubcoreMesh(core_axis_name="c", subcore_axis_name="s",
                                  num_cores=NC, num_subcores=NS)
    scratch = [
        pltpu.VMEM((NPAD,), jnp.int32),
        pltpu.VMEM((NT,), jnp.float32),
        pltpu.VMEM((NT,), jnp.float32),
        pltpu.VMEM((RK * RLEN,), jnp.int32),
        pltpu.VMEM((NT * DK,), jnp.int32),
        pltpu.VMEM((2 * NPAD,), jnp.int32),
        pltpu.VMEM((2 * NPAD,), jnp.int32),
        pltpu.VMEM((2 * NPAD,), jnp.int32),
        pltpu.VMEM((2 * NPAD,), jnp.int32),
        pltpu.VMEM((D4, NT), jnp.float32),
        pltpu.VMEM((DK, NT), jnp.float32),
        pltpu.VMEM((4 * NT,), jnp.float32),
        pltpu.SemaphoreType.DMA,
        pltpu.SemaphoreType.DMA,
    ]

    @functools.partial(
        pl.kernel,
        out_type=jax.ShapeDtypeStruct((OUT * NPAD,), jnp.float32),
        mesh=mesh,
        scratch_types=scratch,
        compiler_params=pltpu.CompilerParams(needs_layout_passes=False,
                                             use_tc_tiling_on_sc=False),
    )
    def sc_kernel(whp_hbm, wdp_hbm, st_hbm, tpk_hbm, ridx_hbm, didx_hbm,
                  out_hbm, tpk_v, s1_v, s2_v, ridx_v, didx_v,
                  qwA, qdA, qwB, qdB, alpha_v, ad_v, outq_v,
                  sem_a, sem_b):
        sid = lax.axis_index("s")
        wid = sid * NC + lax.axis_index("c")
        base = wid * NT

        def _unlo(w):
            return plsc.bitcast(w << 16, jnp.float32)

        def _unhi(w):
            return plsc.bitcast(w & jnp.int32(-65536), jnp.float32)

        pltpu.sync_copy(tpk_hbm, tpk_v)
        pltpu.sync_copy(st_hbm.at[pl.ds(base, NT)], s1_v)
        pltpu.sync_copy(st_hbm.at[pl.ds(2 * NPAD + base, NT)], s2_v)
        pltpu.sync_copy(ridx_hbm.at[pl.ds(wid * (RK * RLEN), RK * RLEN)],
                        ridx_v)
        pltpu.sync_copy(didx_hbm.at[pl.ds(wid * (NT * DK), NT * DK)], didx_v)

        pre = [pltpu.async_copy(whp_hbm.at[pl.ds(0, 2 * NPAD)], qwA, sem_a),
               pltpu.async_copy(wdp_hbm.at[pl.ds(0, 2 * NPAD)], qdA, sem_a),
               pltpu.async_copy(whp_hbm.at[pl.ds(2 * NPAD, 2 * NPAD)], qwB,
                                sem_b),
               pltpu.async_copy(wdp_hbm.at[pl.ds(2 * NPAD, 2 * NPAD)], qdB,
                                sem_b)]

        def attn_group(g, carry):
            goff = g * 16
            s1 = s1_v[pl.ds(goff, 16)]
            es = []
            for d in range(D4):
                acc = None
                for k in range(RK):
                    t = _unlo(plsc.load_gather(
                        tpk_v,
                        [ridx_v[pl.ds(k * RLEN + d * NT + goff, 16)]]))
                    acc = t if acc is None else acc + t
                es.append(_leaky(s1 + (1.0 / RK) * acc))
            m = jnp.maximum(jnp.maximum(es[0], es[1]),
                            jnp.maximum(es[2], es[3]))
            ex = [jnp.exp(e - m) for e in es]
            inv = (1.0 / RK) / ((ex[0] + ex[1]) + (ex[2] + ex[3]))
            for d in range(D4):
                alpha_v[d, pl.ds(goff, 16)] = ex[d] * inv

            s2 = s2_v[pl.ds(goff, 16)]
            doff = (g >> 1) * (DK * DCH) + (g & 1) * 16
            eds = []
            for k in range(DK):
                eds.append(_leaky(s2 + _unhi(plsc.load_gather(
                    tpk_v, [didx_v[pl.ds(doff + k * DCH, 16)]]))))
            m2 = functools.reduce(jnp.maximum, eds)
            ex2 = [jnp.exp(e - m2) for e in eds]
            inv2 = 1.0 / functools.reduce(lambda a, b: a + b, ex2)
            for k in range(DK):
                ad_v[k, pl.ds(goff, 16)] = ex2[k] * inv2
            return carry

        lax.fori_loop(0, NG, attn_group, 0)

        def make_quad_compute(col_wh, col_wd):
            def gbody(g, carry):
                goff = g * 16
                nloc = goff + lax.iota(jnp.int32, 16)
                als = [alpha_v[d, pl.ds(goff, 16)] for d in range(D4)]
                accR = [None] * 4
                for d in range(D4):
                    ts = [None] * 4
                    for k in range(RK):
                        iv = ridx_v[pl.ds(k * RLEN + d * NT + goff, 16)]
                        w0 = plsc.load_gather(col_wh, [iv])
                        w1 = plsc.load_gather(col_wh, [iv + NPAD])
                        for j, part in enumerate(
                                (_unlo(w0), _unhi(w0), _unlo(w1), _unhi(w1))):
                            ts[j] = part if ts[j] is None else ts[j] + part
                    for j in range(4):
                        wv = als[d] * ts[j]
                        accR[j] = wv if accR[j] is None else accR[j] + wv
                doff = (g >> 1) * (DK * DCH) + (g & 1) * 16
                accD = [None] * 4
                for k in range(DK):
                    iv = didx_v[pl.ds(doff + k * DCH, 16)]
                    w0 = plsc.load_gather(col_wd, [iv])
                    w1 = plsc.load_gather(col_wd, [iv + NPAD])
                    adk = ad_v[k, pl.ds(goff, 16)]
                    for j, part in enumerate(
                            (_unlo(w0), _unhi(w0), _unlo(w1), _unhi(w1))):
                        wv = adk * part
                        accD[j] = wv if accD[j] is None else accD[j] + wv
                for j in range(4):
                    val = 0.5 * (_sigmoid(accR[j]) + _sigmoid(accD[j]))
                    plsc.store_scatter(outq_v, [j * NT + nloc], val)
                return carry

            return gbody

        gbody_A = make_quad_compute(qwA, qdA)
        gbody_B = make_quad_compute(qwB, qdB)

        def _flush(q):
            for j in range(4):
                pltpu.sync_copy(
                    outq_v.at[pl.ds(j * NT, NT)],
                    out_hbm.at[pl.ds((4 * q + j) * NPAD + base, NT)])

        def pair_body(p, carry):
            qA = 2 * p
            qB = 2 * p + 1
            pre[0].wait()
            pre[1].wait()
            lax.fori_loop(0, NG, gbody_A, 0)
            _flush(qA)

            @pl.when(p + 1 < QI)
            def _next_a():
                pltpu.async_copy(
                    whp_hbm.at[pl.ds(2 * (qA + 2) * NPAD, 2 * NPAD)],
                    qwA, sem_a)
                pltpu.async_copy(
                    wdp_hbm.at[pl.ds(2 * (qA + 2) * NPAD, 2 * NPAD)],
                    qdA, sem_a)

            pre[2].wait()
            pre[3].wait()
            lax.fori_loop(0, NG, gbody_B, 0)
            _flush(qB)

            @pl.when(p + 1 < QI)
            def _next_b():
                pltpu.async_copy(
                    whp_hbm.at[pl.ds(2 * (qB + 2) * NPAD, 2 * NPAD)],
                    qwB, sem_b)
                pltpu.async_copy(
                    wdp_hbm.at[pl.ds(2 * (qB + 2) * NPAD, 2 * NPAD)],
                    qdB, sem_b)

            return carry

        lax.fori_loop(0, QI, pair_body, 0)

    return sc_kernel


_sc_kernel = _make_sc_kernel()


def kernel(h, W_ref, a_ref, W_dir, a_dir, ref_neighbors, dir_neighbors):
    h_pad = jnp.pad(h, ((0, NPAD - N), (0, 0)))
    Wc = jnp.concatenate([W_ref, W_dir], axis=1)
    a1 = a_ref[:OUT, 0]
    a2 = a_ref[OUT:, 0]
    b1 = a_dir[:OUT, 0]
    b2 = a_dir[OUT:, 0]
    z = jnp.zeros((OUT,), jnp.float32)
    Bmat = jnp.stack(
        [jnp.concatenate([a1, z]), jnp.concatenate([a2, z]),
         jnp.concatenate([z, b1]), jnp.concatenate([z, b2]),
         jnp.zeros((2 * OUT,), jnp.float32), jnp.zeros((2 * OUT,), jnp.float32),
         jnp.zeros((2 * OUT,), jnp.float32), jnp.zeros((2 * OUT,), jnp.float32)],
        axis=1)

    whT, whdT, st = _tc_project(h_pad, Wc, Bmat)

    rn = jnp.pad(ref_neighbors.astype(jnp.int32),
                 ((0, NPAD - N), (0, 0), (0, 0)))
    ridx = rn.reshape(NW, NT, D4, RK).transpose(0, 3, 2, 1).reshape(-1)
    dn = jnp.pad(dir_neighbors.astype(jnp.int32), ((0, NPAD - N), (0, 0)))
    didx = dn.reshape(NW, NDCH, DCH, DK).transpose(0, 1, 3, 2).reshape(-1)

    def _pack2(a, b):
        return lax.bitcast_convert_type(
            jnp.stack([a.astype(jnp.bfloat16), b.astype(jnp.bfloat16)],
                      axis=-1), jnp.int32)

    def _pack_pairs(t):
        return _pack2(t[0::2], t[1::2]).reshape(-1)

    tpk = _pack2(st[1], st[3])

    out_pad = _sc_kernel(_pack_pairs(whT), _pack_pairs(whdT),
                         st.reshape(8 * NPAD), tpk, ridx, didx)
    return jnp.transpose(out_pad.reshape(OUT, NPAD))[:N]
